# baseline (speedup 1.0000x reference)
import functools

import jax
import jax.numpy as jnp
from jax import lax
from jax.experimental import pallas as pl
from jax.experimental.pallas import tpu as pltpu
from jax.experimental.pallas import tpu_sc as plsc

NC, NS, LANES = 2, 16, 16
NW = NC * NS
EPS = 1e-5
BB = 128


def _repack_table(table_t):
    d, v = table_t.shape
    cols = 2048
    grid = (v + cols - 1) // cols

    def body(x_ref, eye_ref, o_ref):
        t = lax.dot_general(x_ref[...], eye_ref[...],
                            (((0,), (0,)), ((), ())),
                            precision=lax.Precision.HIGHEST)
        o_ref[...] = jnp.concatenate([t, t], axis=1)

    return pl.pallas_call(
        body,
        grid=(grid,),
        in_specs=[pl.BlockSpec((d, cols), lambda i: (0, i)),
                  pl.BlockSpec((d, d), lambda i: (0, 0))],
        out_specs=pl.BlockSpec((cols, 2 * d), lambda i: (i, 0)),
        out_shape=jax.ShapeDtypeStruct((v, 2 * d), jnp.float32),
    )(table_t, jnp.eye(d, dtype=jnp.float32))


def _make_encoder(batch, seq_len, d_model, pos_rows):
    assert d_model == 64 and batch % (NW * BB) == 0 and seq_len % 2 == 0
    n_per_w = seq_len * BB
    mesh = plsc.VectorSubcoreMesh(core_axis_name="c", subcore_axis_name="s")

    @functools.partial(
        pl.kernel,
        out_type=jax.ShapeDtypeStruct((seq_len, d_model, batch), jnp.float32),
        mesh=mesh,
        compiler_params=pltpu.CompilerParams(needs_layout_passes=False),
        scratch_types=[
            pltpu.VMEM((n_per_w,), jnp.int32),
            pltpu.VMEM((BB, 2 * d_model), jnp.float32),
            pltpu.VMEM((BB, 2 * d_model), jnp.float32),
            pltpu.VMEM((BB, 2 * d_model), jnp.float32),
            pltpu.VMEM((BB, 2 * d_model), jnp.float32),
            pltpu.VMEM((d_model, BB), jnp.float32),
            pltpu.VMEM((d_model, BB), jnp.float32),
            pltpu.VMEM((pos_rows * d_model,), jnp.float32),
            pltpu.VMEM((d_model,), jnp.float32),
            pltpu.VMEM((d_model,), jnp.float32),
            pltpu.SemaphoreType.DMA,
            pltpu.SemaphoreType.DMA,
            pltpu.SemaphoreType.DMA,
            pltpu.SemaphoreType.DMA,
            pltpu.SemaphoreType.DMA,
            pltpu.SemaphoreType.DMA,
        ],
    )
    def enc(tok_hbm, table_hbm, pos_hbm, gamma_hbm, beta_hbm, out_hbm,
            idx_v, rows_0, rows_1, rows_2, rows_3, out_a, out_b, pos_v,
            gamma_v, beta_v, sem_0, sem_1, sem_2, sem_3, semw_a, semw_b):
        wid = lax.axis_index("s") * NC + lax.axis_index("c")
        row0 = wid * n_per_w
        b0 = wid * BB
        pltpu.sync_copy(tok_hbm.at[pl.ds(row0, n_per_w)], idx_v)
        pltpu.sync_copy(pos_hbm, pos_v)
        pltpu.sync_copy(gamma_hbm, gamma_v)
        pltpu.sync_copy(beta_hbm, beta_v)
        iota = lax.iota(jnp.int32, LANES)
        nbg = BB // LANES
        rowvec = [bg * LANES + iota for bg in range(nbg)]

        rows = (rows_0, rows_1, rows_2, rows_3)
        outs = (out_a, out_b)
        sems = (sem_0, sem_1, sem_2, sem_3)
        semws = (semw_a, semw_b)
        depth = len(rows)

        def start_gather(l, buf, sem):
            pltpu.async_copy(
                table_hbm.at[idx_v.at[pl.ds(l * BB, BB)]], buf, sem)

        def out_slice(l):
            return out_hbm.at[l, :, pl.ds(pl.multiple_of(b0, 8), BB)]

        def process(l, b):
            pltpu.make_async_copy(
                table_hbm.at[idx_v.at[pl.ds(l * BB, BB)]],
                rows[b], sems[b]).wait()
            rv, ov = rows[b], outs[b % 2]

            @pl.when(l >= 2)
            def _():
                pltpu.make_async_copy(
                    ov, out_slice(l - 2), semws[b % 2]).wait()

            pbase = l * d_model

            def pass1(j, carry):
                acc = list(carry)
                jv = jnp.broadcast_to(j, (LANES,))
                pj = plsc.load_gather(pos_v, [pbase + jv])
                for bg in range(nbg):
                    col = plsc.load_gather(rv, [rowvec[bg], jv])
                    x = col + pj
                    acc[2 * bg] = acc[2 * bg] + x
                    acc[2 * bg + 1] = acc[2 * bg + 1] + x * x
                return tuple(acc)

            zero = jnp.zeros((LANES,), jnp.float32)
            acc = pl.loop(0, d_model, init_carry=(zero,) * (2 * nbg),
                          unroll=8)(pass1)

            mean, rstd = [], []
            for bg in range(nbg):
                m = acc[2 * bg] * (1.0 / d_model)
                vv = acc[2 * bg + 1] * (1.0 / d_model) - m * m + EPS
                iv = plsc.bitcast(vv, jnp.int32)
                y = plsc.bitcast(
                    jnp.int32(0x5F3759DF)
                    - lax.shift_right_logical(iv, 1), jnp.float32)
                y = y * (1.5 - 0.5 * vv * y * y)
                y = y * (1.5 - 0.5 * vv * y * y)
                mean.append(m)
                rstd.append(y)

            @pl.loop(0, d_model, unroll=8)
            def pass2(j):
                jv = jnp.broadcast_to(j, (LANES,))
                pj = plsc.load_gather(pos_v, [pbase + jv])
                gj = plsc.load_gather(gamma_v, [jv])
                bj = plsc.load_gather(beta_v, [jv])
                for bg in range(nbg):
                    col = plsc.load_gather(rv, [rowvec[bg], jv])
                    t = col + (pj - mean[bg])
                    ov[j, pl.ds(bg * LANES, LANES)] = t * (rstd[bg] * gj) + bj

            pltpu.async_copy(ov, out_slice(l), semws[b % 2])

            @pl.when(l + depth < seq_len)
            def _():
                start_gather(l + depth, rv, sems[b])

        for b in range(depth):
            start_gather(b, rows[b], sems[b])

        main = seq_len - seq_len % depth

        @pl.loop(0, main, step=depth)
        def _grp(l):
            for b in range(depth):
                process(l + b, b)

        for l in range(main, seq_len):
            process(l, l % depth)

        for b, l in ((0, seq_len - 2), (1, seq_len - 1)):
            pltpu.make_async_copy(outs[b], out_slice(l), semws[b]).wait()

    return enc


def kernel(tokens, token_table, pos_table, gamma, beta):
    batch, seq_len = tokens.shape
    v, d = token_table.shape
    table2 = _repack_table(token_table.T)
    tokw = (tokens.astype(jnp.int32).T
            .reshape(seq_len, batch // BB, BB)
            .transpose(1, 0, 2)
            .reshape(batch * seq_len))
    posf = pos_table.reshape(-1).astype(jnp.float32)
    enc = _make_encoder(batch, seq_len, d, pos_table.shape[0])
    out_t = enc(tokw, table2, posf, gamma, beta)
    return jnp.transpose(out_t, (2, 0, 1))


# speedup vs baseline: 1.4610x; 1.4610x over previous
import functools

import jax
import jax.numpy as jnp
from jax import lax
from jax.experimental import pallas as pl
from jax.experimental.pallas import tpu as pltpu
from jax.experimental.pallas import tpu_sc as plsc

NC, NS, LANES = 2, 16, 16
NW = NC * NS
EPS = 1e-5
BB = 128


def _repack_table(table_t):
    d, v = table_t.shape
    cols = 2048
    grid = (v + cols - 1) // cols

    def body(x_ref, o_ref):
        t = x_ref[...].T
        o_ref[...] = jnp.concatenate([t, t], axis=1)

    return pl.pallas_call(
        body,
        grid=(grid,),
        in_specs=[pl.BlockSpec((d, cols), lambda i: (0, i))],
        out_specs=pl.BlockSpec((cols, 2 * d), lambda i: (i, 0)),
        out_shape=jax.ShapeDtypeStruct((v, 2 * d), jnp.float32),
    )(table_t)


def _make_encoder(batch, seq_len, d_model, pos_rows):
    assert d_model == 64 and batch % (NW * BB) == 0 and seq_len % 2 == 0
    n_per_w = seq_len * BB
    mesh = plsc.VectorSubcoreMesh(core_axis_name="c", subcore_axis_name="s")

    @functools.partial(
        pl.kernel,
        out_type=jax.ShapeDtypeStruct((seq_len, d_model, batch), jnp.float32),
        mesh=mesh,
        compiler_params=pltpu.CompilerParams(needs_layout_passes=False),
        scratch_types=[
            pltpu.VMEM((n_per_w,), jnp.int32),
            pltpu.VMEM((BB, 2 * d_model), jnp.float32),
            pltpu.VMEM((BB, 2 * d_model), jnp.float32),
            pltpu.VMEM((BB, 2 * d_model), jnp.float32),
            pltpu.VMEM((BB, 2 * d_model), jnp.float32),
            pltpu.VMEM((d_model, BB), jnp.float32),
            pltpu.VMEM((d_model, BB), jnp.float32),
            pltpu.VMEM((BB * (d_model + 1),), jnp.float32),
            pltpu.VMEM((pos_rows * d_model,), jnp.float32),
            pltpu.VMEM((d_model,), jnp.float32),
            pltpu.VMEM((d_model,), jnp.float32),
            pltpu.SemaphoreType.DMA,
            pltpu.SemaphoreType.DMA,
            pltpu.SemaphoreType.DMA,
            pltpu.SemaphoreType.DMA,
            pltpu.SemaphoreType.DMA,
            pltpu.SemaphoreType.DMA,
        ],
    )
    def enc(tok_hbm, table_hbm, pos_hbm, gamma_hbm, beta_hbm, out_hbm,
            idx_v, rows_0, rows_1, rows_2, rows_3, out_a, out_b, xbuf, pos_v,
            gamma_v, beta_v, sem_0, sem_1, sem_2, sem_3, semw_a, semw_b):
        wid = lax.axis_index("s") * NC + lax.axis_index("c")
        row0 = wid * n_per_w
        b0 = wid * BB
        pltpu.sync_copy(tok_hbm.at[pl.ds(row0, n_per_w)], idx_v)
        pltpu.sync_copy(pos_hbm, pos_v)
        pltpu.sync_copy(gamma_hbm, gamma_v)
        pltpu.sync_copy(beta_hbm, beta_v)
        iota = lax.iota(jnp.int32, LANES)
        nbg = BB // LANES
        rowvec = [bg * LANES + iota for bg in range(nbg)]

        rows = (rows_0, rows_1, rows_2, rows_3)
        outs = (out_a, out_b)
        sems = (sem_0, sem_1, sem_2, sem_3)
        semws = (semw_a, semw_b)
        depth = len(rows)

        def start_gather(l, buf, sem):
            pltpu.async_copy(
                table_hbm.at[idx_v.at[pl.ds(l * BB, BB)]], buf, sem)

        def out_slice(l):
            return out_hbm.at[l, :, pl.ds(pl.multiple_of(b0, 8), BB)]

        def process(l, b):
            pltpu.make_async_copy(
                table_hbm.at[idx_v.at[pl.ds(l * BB, BB)]],
                rows[b], sems[b]).wait()
            rv, ov = rows[b], outs[b % 2]

            @pl.when(l >= 2)
            def _():
                pltpu.make_async_copy(
                    ov, out_slice(l - 2), semws[b % 2]).wait()

            pbase = l * d_model
            p4 = [pos_v[pl.ds(pbase + c * LANES, LANES)]
                  for c in range(d_model // LANES)]
            stride = d_model + 1

            @pl.loop(0, BB, unroll=4)
            def _stage(r):
                for c in range(d_model // LANES):
                    xbuf[pl.ds(r * stride + c * LANES, LANES)] = (
                        rv[r, pl.ds(c * LANES, LANES)] + p4[c])

            @pl.when(l + depth < seq_len)
            def _():
                start_gather(l + depth, rv, sems[b])

            fbase = [(bg * LANES + iota) * stride for bg in range(nbg)]

            def pass1(j, carry):
                acc = list(carry)
                jv = jnp.broadcast_to(j, (LANES,))
                for bg in range(nbg):
                    x = plsc.load_gather(xbuf, [fbase[bg] + jv])
                    acc[2 * bg] = acc[2 * bg] + x
                    acc[2 * bg + 1] = acc[2 * bg + 1] + x * x
                return tuple(acc)

            zero = jnp.zeros((LANES,), jnp.float32)
            acc = pl.loop(0, d_model, init_carry=(zero,) * (2 * nbg),
                          unroll=8)(pass1)

            mean, rstd = [], []
            for bg in range(nbg):
                m = acc[2 * bg] * (1.0 / d_model)
                vv = acc[2 * bg + 1] * (1.0 / d_model) - m * m + EPS
                iv = plsc.bitcast(vv, jnp.int32)
                y = plsc.bitcast(
                    jnp.int32(0x5F3759DF)
                    - lax.shift_right_logical(iv, 1), jnp.float32)
                y = y * (1.5 - 0.5 * vv * y * y)
                y = y * (1.5 - 0.5 * vv * y * y)
                mean.append(m)
                rstd.append(y)

            @pl.loop(0, d_model, unroll=8)
            def pass2(j):
                jv = jnp.broadcast_to(j, (LANES,))
                gj = plsc.load_gather(gamma_v, [jv])
                bj = plsc.load_gather(beta_v, [jv])
                for bg in range(nbg):
                    x = plsc.load_gather(xbuf, [fbase[bg] + jv])
                    ov[j, pl.ds(bg * LANES, LANES)] = (
                        (x - mean[bg]) * (rstd[bg] * gj) + bj)

            pltpu.async_copy(ov, out_slice(l), semws[b % 2])

        for b in range(depth):
            start_gather(b, rows[b], sems[b])

        main = seq_len - seq_len % depth

        @pl.loop(0, main, step=depth)
        def _grp(l):
            for b in range(depth):
                process(l + b, b)

        for l in range(main, seq_len):
            process(l, l % depth)

        for b, l in ((0, seq_len - 2), (1, seq_len - 1)):
            pltpu.make_async_copy(outs[b], out_slice(l), semws[b]).wait()

    return enc


def kernel(tokens, token_table, pos_table, gamma, beta):
    batch, seq_len = tokens.shape
    v, d = token_table.shape
    table2 = _repack_table(token_table.T)
    tokw = (tokens.astype(jnp.int32).T
            .reshape(seq_len, batch // BB, BB)
            .transpose(1, 0, 2)
            .reshape(batch * seq_len))
    posf = pos_table.reshape(-1).astype(jnp.float32)
    enc = _make_encoder(batch, seq_len, d, pos_table.shape[0])
    out_t = enc(tokw, table2, posf, gamma, beta)
    return jnp.transpose(out_t, (2, 0, 1))


# speedup vs baseline: 1.8821x; 1.2882x over previous
import functools

import jax
import jax.numpy as jnp
from jax import lax
from jax.experimental import pallas as pl
from jax.experimental.pallas import tpu as pltpu
from jax.experimental.pallas import tpu_sc as plsc

NC, NS, LANES = 2, 16, 16
NW = NC * NS
EPS = 1e-5
BB = 128


def _repack_table(table_t3):
    d, v = table_t3.shape
    cols = 8192
    grid = (v + cols - 1) // cols

    def body(x_ref, o_ref):
        t = x_ref[...].T
        o_ref[...] = jnp.concatenate([t, t], axis=1)

    return pl.pallas_call(
        body,
        grid=(grid,),
        in_specs=[pl.BlockSpec((d, cols), lambda i: (0, i))],
        out_specs=pl.BlockSpec((cols, 2 * d), lambda i: (i, 0)),
        out_shape=jax.ShapeDtypeStruct((v, 2 * d), jnp.float32),
    )(table_t3)


def _make_encoder(batch, seq_len, d_model, pos_rows):
    assert d_model == 64 and batch % (NW * BB) == 0 and seq_len % 2 == 0
    n_per_w = seq_len * BB
    mesh = plsc.VectorSubcoreMesh(core_axis_name="c", subcore_axis_name="s")

    @functools.partial(
        pl.kernel,
        out_type=jax.ShapeDtypeStruct((seq_len, d_model, batch), jnp.float32),
        mesh=mesh,
        compiler_params=pltpu.CompilerParams(needs_layout_passes=False),
        scratch_types=[
            pltpu.VMEM((n_per_w,), jnp.int32),
            pltpu.VMEM((BB, 2 * d_model), jnp.float32),
            pltpu.VMEM((BB, 2 * d_model), jnp.float32),
            pltpu.VMEM((BB, 2 * d_model), jnp.float32),
            pltpu.VMEM((BB, 2 * d_model), jnp.float32),
            pltpu.VMEM((d_model, BB), jnp.float32),
            pltpu.VMEM((d_model, BB), jnp.float32),
            pltpu.VMEM((BB * (d_model + 1),), jnp.float32),
            pltpu.VMEM((pos_rows * d_model,), jnp.float32),
            pltpu.VMEM((d_model,), jnp.float32),
            pltpu.VMEM((d_model,), jnp.float32),
            pltpu.SemaphoreType.DMA,
            pltpu.SemaphoreType.DMA,
            pltpu.SemaphoreType.DMA,
            pltpu.SemaphoreType.DMA,
            pltpu.SemaphoreType.DMA,
            pltpu.SemaphoreType.DMA,
        ],
    )
    def enc(tok_hbm, table_hbm, pos_hbm, gamma_hbm, beta_hbm, out_hbm,
            idx_v, rows_0, rows_1, rows_2, rows_3, out_a, out_b, xbuf, pos_v,
            gamma_v, beta_v, sem_0, sem_1, sem_2, sem_3, semw_a, semw_b):
        wid = lax.axis_index("s") * NC + lax.axis_index("c")
        row0 = wid * n_per_w
        b0 = wid * BB
        pltpu.sync_copy(tok_hbm.at[pl.ds(row0, n_per_w)], idx_v)
        pltpu.sync_copy(pos_hbm, pos_v)
        pltpu.sync_copy(gamma_hbm, gamma_v)
        pltpu.sync_copy(beta_hbm, beta_v)
        iota = lax.iota(jnp.int32, LANES)
        nbg = BB // LANES
        rowvec = [bg * LANES + iota for bg in range(nbg)]

        rows = (rows_0, rows_1, rows_2, rows_3)
        outs = (out_a, out_b)
        sems = (sem_0, sem_1, sem_2, sem_3)
        semws = (semw_a, semw_b)
        depth = len(rows)

        def start_gather(l, buf, sem):
            pltpu.async_copy(
                table_hbm.at[idx_v.at[pl.ds(l * BB, BB)]], buf, sem)

        def out_slice(l):
            return out_hbm.at[l, :, pl.ds(pl.multiple_of(b0, 8), BB)]

        def process(l, b):
            pltpu.make_async_copy(
                table_hbm.at[idx_v.at[pl.ds(l * BB, BB)]],
                rows[b], sems[b]).wait()
            rv, ov = rows[b], outs[b % 2]

            @pl.when(l >= 2)
            def _():
                pltpu.make_async_copy(
                    ov, out_slice(l - 2), semws[b % 2]).wait()

            pbase = l * d_model
            p4 = [pos_v[pl.ds(pbase + c * LANES, LANES)]
                  for c in range(d_model // LANES)]
            stride = d_model + 1

            @pl.loop(0, BB, unroll=4)
            def _stage(r):
                for c in range(d_model // LANES):
                    xbuf[pl.ds(r * stride + c * LANES, LANES)] = (
                        rv[r, pl.ds(c * LANES, LANES)] + p4[c])

            @pl.when(l + depth < seq_len)
            def _():
                start_gather(l + depth, rv, sems[b])

            fbase = [(bg * LANES + iota) * stride for bg in range(nbg)]

            def pass1(j, carry):
                acc = list(carry)
                jv = jnp.broadcast_to(j, (LANES,))
                for bg in range(nbg):
                    x = plsc.load_gather(xbuf, [fbase[bg] + jv])
                    acc[2 * bg] = acc[2 * bg] + x
                    acc[2 * bg + 1] = acc[2 * bg + 1] + x * x
                return tuple(acc)

            zero = jnp.zeros((LANES,), jnp.float32)
            acc = pl.loop(0, d_model, init_carry=(zero,) * (2 * nbg),
                          unroll=8)(pass1)

            mean, rstd = [], []
            for bg in range(nbg):
                m = acc[2 * bg] * (1.0 / d_model)
                vv = acc[2 * bg + 1] * (1.0 / d_model) - m * m + EPS
                iv = plsc.bitcast(vv, jnp.int32)
                y = plsc.bitcast(
                    jnp.int32(0x5F3759DF)
                    - lax.shift_right_logical(iv, 1), jnp.float32)
                y = y * (1.5 - 0.5 * vv * y * y)
                y = y * (1.5 - 0.5 * vv * y * y)
                mean.append(m)
                rstd.append(y)

            @pl.loop(0, d_model, unroll=8)
            def pass2(j):
                jv = jnp.broadcast_to(j, (LANES,))
                gj = plsc.load_gather(gamma_v, [jv])
                bj = plsc.load_gather(beta_v, [jv])
                for bg in range(nbg):
                    x = plsc.load_gather(xbuf, [fbase[bg] + jv])
                    ov[j, pl.ds(bg * LANES, LANES)] = (
                        (x - mean[bg]) * (rstd[bg] * gj) + bj)

            pltpu.async_copy(ov, out_slice(l), semws[b % 2])

        for b in range(depth):
            start_gather(b, rows[b], sems[b])

        main = seq_len - seq_len % depth

        @pl.loop(0, main, step=depth)
        def _grp(l):
            for b in range(depth):
                process(l + b, b)

        for l in range(main, seq_len):
            process(l, l % depth)

        for b, l in ((0, seq_len - 2), (1, seq_len - 1)):
            pltpu.make_async_copy(outs[b], out_slice(l), semws[b]).wait()

    return enc


def kernel(tokens, token_table, pos_table, gamma, beta):
    batch, seq_len = tokens.shape
    v, d = token_table.shape
    table2 = _repack_table(token_table.T)
    tokw = (tokens.astype(jnp.int32).T
            .reshape(seq_len, batch // BB, BB)
            .transpose(1, 0, 2)
            .reshape(batch * seq_len))
    posf = pos_table.reshape(-1).astype(jnp.float32)
    enc = _make_encoder(batch, seq_len, d, pos_table.shape[0])
    out_t = enc(tokw, table2, posf, gamma, beta)
    return jnp.transpose(out_t, (2, 0, 1))


# speedup vs baseline: 1.9810x; 1.0526x over previous
import functools

import jax
import jax.numpy as jnp
from jax import lax
from jax.experimental import pallas as pl
from jax.experimental.pallas import tpu as pltpu
from jax.experimental.pallas import tpu_sc as plsc

NC, NS, LANES = 2, 16, 16
NW = NC * NS
EPS = 1e-5
BB = 128


def _repack_table(table_t3):
    d, v = table_t3.shape
    cols = 16384
    grid = (v + cols - 1) // cols

    def body(x_ref, o_ref):
        t = x_ref[...].T
        o_ref[...] = jnp.concatenate([t, t], axis=1)

    return pl.pallas_call(
        body,
        grid=(grid,),
        in_specs=[pl.BlockSpec((d, cols), lambda i: (0, i))],
        out_specs=pl.BlockSpec((cols, 2 * d), lambda i: (i, 0)),
        out_shape=jax.ShapeDtypeStruct((v, 2 * d), jnp.float32),
    )(table_t3)


def _make_encoder(batch, seq_len, d_model, pos_rows):
    assert d_model == 64 and batch % (NW * BB) == 0 and seq_len % 2 == 0
    n_per_w = seq_len * BB
    mesh = plsc.VectorSubcoreMesh(core_axis_name="c", subcore_axis_name="s")

    @functools.partial(
        pl.kernel,
        out_type=jax.ShapeDtypeStruct((seq_len, d_model, batch), jnp.float32),
        mesh=mesh,
        compiler_params=pltpu.CompilerParams(needs_layout_passes=False),
        scratch_types=[
            pltpu.VMEM((n_per_w,), jnp.int32),
            pltpu.VMEM((BB, 2 * d_model), jnp.float32),
            pltpu.VMEM((BB, 2 * d_model), jnp.float32),
            pltpu.VMEM((BB, 2 * d_model), jnp.float32),
            pltpu.VMEM((BB, 2 * d_model), jnp.float32),
            pltpu.VMEM((d_model, BB), jnp.float32),
            pltpu.VMEM((d_model, BB), jnp.float32),
            pltpu.VMEM((BB * (d_model + 1),), jnp.float32),
            pltpu.VMEM((pos_rows * d_model,), jnp.float32),
            pltpu.VMEM((d_model,), jnp.float32),
            pltpu.VMEM((d_model,), jnp.float32),
            pltpu.SemaphoreType.DMA,
            pltpu.SemaphoreType.DMA,
            pltpu.SemaphoreType.DMA,
            pltpu.SemaphoreType.DMA,
            pltpu.SemaphoreType.DMA,
            pltpu.SemaphoreType.DMA,
        ],
    )
    def enc(tok_hbm, table_hbm, pos_hbm, gamma_hbm, beta_hbm, out_hbm,
            idx_v, rows_0, rows_1, rows_2, rows_3, out_a, out_b, xbuf, pos_v,
            gamma_v, beta_v, sem_0, sem_1, sem_2, sem_3, semw_a, semw_b):
        wid = lax.axis_index("s") * NC + lax.axis_index("c")
        row0 = wid * n_per_w
        b0 = wid * BB
        pltpu.sync_copy(tok_hbm.at[pl.ds(row0, n_per_w)], idx_v)
        pltpu.sync_copy(pos_hbm, pos_v)
        pltpu.sync_copy(gamma_hbm, gamma_v)
        pltpu.sync_copy(beta_hbm, beta_v)
        iota = lax.iota(jnp.int32, LANES)
        nbg = BB // LANES
        rowvec = [bg * LANES + iota for bg in range(nbg)]

        rows = (rows_0, rows_1, rows_2, rows_3)
        outs = (out_a, out_b)
        sems = (sem_0, sem_1, sem_2, sem_3)
        semws = (semw_a, semw_b)
        depth = len(rows)

        def start_gather(l, buf, sem):
            pltpu.async_copy(
                table_hbm.at[idx_v.at[pl.ds(l * BB, BB)]], buf, sem)

        def out_slice(l):
            return out_hbm.at[l, :, pl.ds(pl.multiple_of(b0, 8), BB)]

        def process(l, b):
            pltpu.make_async_copy(
                table_hbm.at[idx_v.at[pl.ds(l * BB, BB)]],
                rows[b], sems[b]).wait()
            rv, ov = rows[b], outs[b % 2]

            @pl.when(l >= 2)
            def _():
                pltpu.make_async_copy(
                    ov, out_slice(l - 2), semws[b % 2]).wait()

            pbase = l * d_model
            p4 = [pos_v[pl.ds(pbase + c * LANES, LANES)]
                  for c in range(d_model // LANES)]
            stride = d_model + 1

            @pl.loop(0, BB, unroll=4)
            def _stage(r):
                for c in range(d_model // LANES):
                    xbuf[pl.ds(r * stride + c * LANES, LANES)] = (
                        rv[r, pl.ds(c * LANES, LANES)] + p4[c])

            @pl.when(l + depth < seq_len)
            def _():
                start_gather(l + depth, rv, sems[b])

            fbase = [(bg * LANES + iota) * stride for bg in range(nbg)]

            def pass1(j, carry):
                acc = list(carry)
                jv = jnp.broadcast_to(j, (LANES,))
                for bg in range(nbg):
                    x = plsc.load_gather(xbuf, [fbase[bg] + jv])
                    acc[2 * bg] = acc[2 * bg] + x
                    acc[2 * bg + 1] = acc[2 * bg + 1] + x * x
                return tuple(acc)

            zero = jnp.zeros((LANES,), jnp.float32)
            acc = pl.loop(0, d_model, init_carry=(zero,) * (2 * nbg),
                          unroll=8)(pass1)

            mean, rstd = [], []
            for bg in range(nbg):
                m = acc[2 * bg] * (1.0 / d_model)
                vv = acc[2 * bg + 1] * (1.0 / d_model) - m * m + EPS
                iv = plsc.bitcast(vv, jnp.int32)
                y = plsc.bitcast(
                    jnp.int32(0x5F3759DF)
                    - lax.shift_right_logical(iv, 1), jnp.float32)
                y = y * (1.5 - 0.5 * vv * y * y)
                y = y * (1.5 - 0.5 * vv * y * y)
                mean.append(m)
                rstd.append(y)

            @pl.loop(0, d_model, unroll=8)
            def pass2(j):
                jv = jnp.broadcast_to(j, (LANES,))
                gj = plsc.load_gather(gamma_v, [jv])
                bj = plsc.load_gather(beta_v, [jv])
                for bg in range(nbg):
                    x = plsc.load_gather(xbuf, [fbase[bg] + jv])
                    ov[j, pl.ds(bg * LANES, LANES)] = (
                        (x - mean[bg]) * (rstd[bg] * gj) + bj)

            pltpu.async_copy(ov, out_slice(l), semws[b % 2])

        for b in range(depth):
            start_gather(b, rows[b], sems[b])

        main = seq_len - seq_len % depth

        @pl.loop(0, main, step=depth)
        def _grp(l):
            for b in range(depth):
                process(l + b, b)

        for l in range(main, seq_len):
            process(l, l % depth)

        for b, l in ((0, seq_len - 2), (1, seq_len - 1)):
            pltpu.make_async_copy(outs[b], out_slice(l), semws[b]).wait()

    return enc


def kernel(tokens, token_table, pos_table, gamma, beta):
    batch, seq_len = tokens.shape
    v, d = token_table.shape
    table2 = _repack_table(token_table.T)
    tokw = (tokens.astype(jnp.int32).T
            .reshape(seq_len, batch // BB, BB)
            .transpose(1, 0, 2)
            .reshape(batch * seq_len))
    posf = pos_table.reshape(-1).astype(jnp.float32)
    enc = _make_encoder(batch, seq_len, d, pos_table.shape[0])
    out_t = enc(tokw, table2, posf, gamma, beta)
    return jnp.transpose(out_t, (2, 0, 1))


# speedup vs baseline: 1.9825x; 1.0007x over previous
import functools

import jax
import jax.numpy as jnp
from jax import lax
from jax.experimental import pallas as pl
from jax.experimental.pallas import tpu as pltpu
from jax.experimental.pallas import tpu_sc as plsc

NC, NS, LANES = 2, 16, 16
NW = NC * NS
EPS = 1e-5
BB = 128


def _repack_table(table_t):
    d, v = table_t.shape
    cols = 16384
    grid = (v + cols - 1) // cols

    def body(x_ref, o_ref):
        t = x_ref[...].T
        o_ref[...] = jnp.concatenate([t, t], axis=1)

    return pl.pallas_call(
        body,
        grid=(grid,),
        in_specs=[pl.BlockSpec((d, cols), lambda i: (0, i))],
        out_specs=pl.BlockSpec((cols, 2 * d), lambda i: (i, 0)),
        out_shape=jax.ShapeDtypeStruct((v, 2 * d), jnp.float32),
    )(table_t)


def _make_encoder(batch, seq_len, d_model, pos_rows):
    assert d_model == 64 and batch % (NW * BB) == 0 and seq_len % 2 == 0
    n_per_w = seq_len * BB
    mesh = plsc.VectorSubcoreMesh(core_axis_name="c", subcore_axis_name="s")

    @functools.partial(
        pl.kernel,
        out_type=jax.ShapeDtypeStruct((seq_len, d_model, batch), jnp.float32),
        mesh=mesh,
        compiler_params=pltpu.CompilerParams(needs_layout_passes=False),
        scratch_types=[
            pltpu.VMEM((n_per_w,), jnp.int32),
            pltpu.VMEM((BB, 2 * d_model), jnp.float32),
            pltpu.VMEM((BB, 2 * d_model), jnp.float32),
            pltpu.VMEM((BB, 2 * d_model), jnp.float32),
            pltpu.VMEM((BB, 2 * d_model), jnp.float32),
            pltpu.VMEM((d_model, BB), jnp.float32),
            pltpu.VMEM((d_model, BB), jnp.float32),
            pltpu.VMEM((BB * (d_model + 1),), jnp.float32),
            pltpu.VMEM((pos_rows * d_model,), jnp.float32),
            pltpu.VMEM((d_model,), jnp.float32),
            pltpu.VMEM((d_model,), jnp.float32),
            pltpu.SemaphoreType.DMA,
            pltpu.SemaphoreType.DMA,
            pltpu.SemaphoreType.DMA,
            pltpu.SemaphoreType.DMA,
            pltpu.SemaphoreType.DMA,
            pltpu.SemaphoreType.DMA,
        ],
    )
    def enc(tok_hbm, table_hbm, pos_hbm, gamma_hbm, beta_hbm, out_hbm,
            idx_v, rows_0, rows_1, rows_2, rows_3, out_a, out_b, xbuf, pos_v,
            gamma_v, beta_v, sem_0, sem_1, sem_2, sem_3, semw_a, semw_b):
        wid = lax.axis_index("s") * NC + lax.axis_index("c")
        row0 = wid * n_per_w
        b0 = wid * BB
        pltpu.sync_copy(tok_hbm.at[pl.ds(row0, n_per_w)], idx_v)
        pltpu.sync_copy(pos_hbm, pos_v)
        pltpu.sync_copy(gamma_hbm, gamma_v)
        pltpu.sync_copy(beta_hbm, beta_v)
        iota = lax.iota(jnp.int32, LANES)
        nbg = BB // LANES

        rows = (rows_0, rows_1, rows_2, rows_3)
        outs = (out_a, out_b)
        sems = (sem_0, sem_1, sem_2, sem_3)
        semws = (semw_a, semw_b)
        depth = len(rows)

        def start_gather(l, buf, sem):
            pltpu.async_copy(
                table_hbm.at[idx_v.at[pl.ds(l * BB, BB)]], buf, sem)

        def out_slice(l):
            return out_hbm.at[l, :, pl.ds(pl.multiple_of(b0, 8), BB)]

        def process(l, b):
            pltpu.make_async_copy(
                table_hbm.at[idx_v.at[pl.ds(l * BB, BB)]],
                rows[b], sems[b]).wait()
            rv, ov = rows[b], outs[b % 2]

            @pl.when(l >= 2)
            def _():
                pltpu.make_async_copy(
                    ov, out_slice(l - 2), semws[b % 2]).wait()

            pbase = l * d_model
            p4 = [pos_v[pl.ds(pbase + c * LANES, LANES)]
                  for c in range(d_model // LANES)]
            stride = d_model + 1

            @pl.loop(0, BB, unroll=4)
            def _stage(r):
                for c in range(d_model // LANES):
                    xbuf[pl.ds(r * stride + c * LANES, LANES)] = (
                        rv[r, pl.ds(c * LANES, LANES)] + p4[c])

            @pl.when(l + depth < seq_len)
            def _():
                start_gather(l + depth, rv, sems[b])

            fbase = [(bg * LANES + iota) * stride for bg in range(nbg)]

            def pass1(j, carry):
                acc = list(carry)
                jv = jnp.broadcast_to(j, (LANES,))
                for bg in range(nbg):
                    x = plsc.load_gather(xbuf, [fbase[bg] + jv])
                    acc[2 * bg] = acc[2 * bg] + x
                    acc[2 * bg + 1] = acc[2 * bg + 1] + x * x
                return tuple(acc)

            zero = jnp.zeros((LANES,), jnp.float32)
            acc = pl.loop(0, d_model, init_carry=(zero,) * (2 * nbg),
                          unroll=8)(pass1)

            mean, rstd = [], []
            for bg in range(nbg):
                m = acc[2 * bg] * (1.0 / d_model)
                vv = acc[2 * bg + 1] * (1.0 / d_model) - m * m + EPS
                iv = plsc.bitcast(vv, jnp.int32)
                y = plsc.bitcast(
                    jnp.int32(0x5F3759DF)
                    - lax.shift_right_logical(iv, 1), jnp.float32)
                y = y * (1.5 - 0.5 * vv * y * y)
                y = y * (1.5 - 0.5 * vv * y * y)
                mean.append(m)
                rstd.append(y)

            @pl.loop(0, d_model, unroll=8)
            def pass2(j):
                jv = jnp.broadcast_to(j, (LANES,))
                gj = plsc.load_gather(gamma_v, [jv])
                bj = plsc.load_gather(beta_v, [jv])
                for bg in range(nbg):
                    x = plsc.load_gather(xbuf, [fbase[bg] + jv])
                    ov[j, pl.ds(bg * LANES, LANES)] = (
                        (x - mean[bg]) * (rstd[bg] * gj) + bj)

            pltpu.async_copy(ov, out_slice(l), semws[b % 2])

        for b in range(depth):
            start_gather(b, rows[b], sems[b])

        main = seq_len - seq_len % depth

        @pl.loop(0, main, step=depth)
        def _grp(l):
            for b in range(depth):
                process(l + b, b)

        for l in range(main, seq_len):
            process(l, l % depth)

        for b, l in ((0, seq_len - 2), (1, seq_len - 1)):
            pltpu.make_async_copy(outs[b], out_slice(l), semws[b]).wait()

    return enc


def kernel(tokens, token_table, pos_table, gamma, beta):
    batch, seq_len = tokens.shape
    v, d = token_table.shape
    table2 = _repack_table(token_table.T)
    tokw = (tokens.astype(jnp.int32).T
            .reshape(seq_len, batch // BB, BB)
            .transpose(1, 0, 2)
            .reshape(batch * seq_len))
    posf = pos_table.reshape(-1).astype(jnp.float32)
    enc = _make_encoder(batch, seq_len, d, pos_table.shape[0])
    out_t = enc(tokw, table2, posf, gamma, beta)
    return jnp.transpose(out_t, (2, 0, 1))
